# Initial kernel scaffold; baseline (speedup 1.0000x reference)
import jax
import jax.numpy as jnp
from jax.experimental import pallas as pl


def kernel(x, edge_index, W1_l, b1, W1_r, W2_l, b2, W2_r, Wc, bc):
    raise NotImplementedError("write your pallas kernel here")


# speedup vs baseline: 6.6801x; 6.6801x over previous
import functools

import jax
import jax.numpy as jnp
from jax import lax
from jax.experimental import pallas as pl
from jax.experimental.pallas import tpu as pltpu
from jax.experimental.pallas import tpu_sc as plsc

N = 10000
E = 320000
D = 128
CLS = 64
NC = 2
NS = 16
NW = NC * NS
E_PER_W = E // NW
CHUNK = 200
N_CHUNKS = E_PER_W // CHUNK
N_PAD = 10240
ROWS_PER_TILE = N_PAD // NS
ZROWS = 40
CZROWS = 160
CW = 128


_MESH = plsc.VectorSubcoreMesh(core_axis_name="c", subcore_axis_name="s")


def _build_agg(interpret=False):
  @functools.partial(
    pl.kernel, mesh=_MESH, interpret=interpret,
    out_type=jax.ShapeDtypeStruct((NC, N_PAD, D), jnp.float32),
    scratch_types=[
        pltpu.VMEM((CHUNK,), jnp.int32),
        pltpu.VMEM((CHUNK,), jnp.int32),
        pltpu.VMEM((CHUNK, D), jnp.float32),
        pltpu.VMEM_SHARED((N_PAD, D), jnp.float32),
        pltpu.SemaphoreType.DMA,
    ])
  def _agg(table, src, dst, zrows, out, src_v, dst_v, rows_v, acc, sem):
    cid = lax.axis_index("c")
    sid = lax.axis_index("s")
    wid = sid * NC + cid

    row0 = sid * ROWS_PER_TILE
    pltpu.sync_copy(zrows, acc.at[pl.ds(row0, ROWS_PER_TILE)])
    plsc.subcore_barrier()

    base = wid * E_PER_W

    def step(g, _):
        off = base + g * CHUNK
        pltpu.sync_copy(src.at[pl.ds(off, CHUNK)], src_v)
        pltpu.sync_copy(dst.at[pl.ds(off, CHUNK)], dst_v)
        pltpu.async_copy(table.at[src_v], rows_v, sem).wait()
        pltpu.sync_copy(rows_v, acc.at[dst_v], add=True)
        return 0
    lax.fori_loop(0, N_CHUNKS, step, 0)

    plsc.subcore_barrier()
    pltpu.sync_copy(acc.at[pl.ds(row0, ROWS_PER_TILE)],
                    out.at[cid, pl.ds(row0, ROWS_PER_TILE)])


  return _agg


def _build_cnt(interpret=False):
  @functools.partial(
    pl.kernel, mesh=_MESH, interpret=interpret,
    out_type=jax.ShapeDtypeStruct((NC, N_PAD, CW), jnp.float32),
    scratch_types=[
        pltpu.VMEM((CHUNK,), jnp.int32),
        pltpu.VMEM((CHUNK, CW), jnp.float32),
        pltpu.VMEM_SHARED((N_PAD, CW), jnp.float32),
    ])
  def _cnt(dst, ones, zrows, cnt_out, dst_v, ones_v, cacc):
    cid = lax.axis_index("c")
    sid = lax.axis_index("s")
    wid = sid * NC + cid

    pltpu.sync_copy(ones, ones_v)
    row0 = sid * ROWS_PER_TILE
    pltpu.sync_copy(zrows, cacc.at[pl.ds(row0, ROWS_PER_TILE)])
    plsc.subcore_barrier()

    base = wid * E_PER_W

    def step(g, _):
        pltpu.sync_copy(dst.at[pl.ds(base + g * CHUNK, CHUNK)], dst_v)
        pltpu.sync_copy(ones_v, cacc.at[dst_v], add=True)
        return 0
    lax.fori_loop(0, N_CHUNKS, step, 0)

    plsc.subcore_barrier()
    pltpu.sync_copy(cacc.at[pl.ds(row0, ROWS_PER_TILE)],
                    cnt_out.at[cid, pl.ds(row0, ROWS_PER_TILE)])

  return _cnt


_agg = _build_agg()
_cnt = _build_cnt()

RB = 2000


def _sage_body(p0, p1, c0, c1, h, wl, wr, b, o):
    cnt = jnp.maximum(c0[:, 0:1] + c1[:, 0:1], 1.0)
    mean = (p0[...] + p1[...]) / cnt
    acc = jnp.dot(mean, wl[...], preferred_element_type=jnp.float32)
    acc += jnp.dot(h[...], wr[...], preferred_element_type=jnp.float32)
    o[...] = jnp.maximum(acc + b[...], 0.0)


def _final_body(p0, p1, c0, c1, h, wl, wr, b, wc, bc, o):
    cnt = jnp.maximum(c0[:, 0:1] + c1[:, 0:1], 1.0)
    mean = (p0[...] + p1[...]) / cnt
    acc = jnp.dot(mean, wl[...], preferred_element_type=jnp.float32)
    acc += jnp.dot(h[...], wr[...], preferred_element_type=jnp.float32)
    h2 = jnp.maximum(acc + b[...], 0.0)
    o[...] = jnp.dot(h2, wc[...], preferred_element_type=jnp.float32) + bc[...]


def _row_spec(w):
    return pl.BlockSpec((RB, w), lambda i: (i, 0))


def _full_spec(r, c):
    return pl.BlockSpec((r, c), lambda i: (0, 0))


def _sage_tc(p0, p1, c0, c1, h, wl, wr, b):
    return pl.pallas_call(
        _sage_body,
        grid=(N // RB,),
        in_specs=[_row_spec(D), _row_spec(D), _row_spec(CW), _row_spec(CW),
                  _row_spec(D), _full_spec(D, D), _full_spec(D, D),
                  _full_spec(1, D)],
        out_specs=_row_spec(D),
        out_shape=jax.ShapeDtypeStruct((N, D), jnp.float32),
    )(p0, p1, c0, c1, h, wl, wr, b)


def _final_tc(p0, p1, c0, c1, h, wl, wr, b, wc, bc):
    return pl.pallas_call(
        _final_body,
        grid=(N // RB,),
        in_specs=[_row_spec(D), _row_spec(D), _row_spec(CW), _row_spec(CW),
                  _row_spec(D), _full_spec(D, D), _full_spec(D, D),
                  _full_spec(1, D), _full_spec(D, CLS), _full_spec(1, CLS)],
        out_specs=_row_spec(CLS),
        out_shape=jax.ShapeDtypeStruct((N, CLS), jnp.float32),
    )(p0, p1, c0, c1, h, wl, wr, b, wc, bc)


def kernel(x, edge_index, W1_l, b1, W1_r, W2_l, b2, W2_r, Wc, bc):
    ei = edge_index.astype(jnp.int32)
    src, dst = ei[0], ei[1]
    zrows = jnp.zeros((ROWS_PER_TILE, D), jnp.float32)
    ones = jnp.ones((CHUNK, CW), jnp.float32)
    cnts = _cnt(dst, ones, zrows)
    parts1 = _agg(x, src, dst, zrows)
    c0, c1 = cnts[0], cnts[1]
    h1 = _sage_tc(parts1[0], parts1[1], c0, c1, x,
                  W1_l, W1_r, b1.reshape(1, D))
    parts2 = _agg(h1, src, dst, zrows)
    return _final_tc(parts2[0], parts2[1], c0, c1, h1,
                     W2_l, W2_r, b2.reshape(1, D), Wc, bc.reshape(1, CLS))
